# baseline (speedup 1.0000x reference)
import functools

import jax
import jax.numpy as jnp
from jax import lax
from jax.experimental import pallas as pl
from jax.experimental.pallas import tpu as pltpu
from jax.experimental.pallas import tpu_sc as plsc

_B = 16384
_D = 32
_BLK = 2048
_LANES = 128
_GRP = 2
_DEPTH = 4
_FLUSH = 128


@functools.cache
def _make_sc_gather(nb):
    info = plsc.get_sparse_core_info()
    nc, ns = info.num_cores, info.num_subcores
    nw = nc * ns
    bpw = nb // nw
    ngrp = bpw // _GRP
    fpw = bpw * _D
    mesh = plsc.VectorSubcoreMesh(core_axis_name="c", subcore_axis_name="s")
    f32 = jnp.float32
    i32 = jnp.int32

    slab = pltpu.VMEM((_D, _LANES), f32)
    @functools.partial(
        pl.kernel,
        mesh=mesh,
        out_type=[jax.ShapeDtypeStruct((nb * _D,), f32)] * 4,
        scratch_types=(
            [pltpu.VMEM((bpw,), i32)] * 4
            + [slab] * (_DEPTH * _GRP * 2)
            + [pltpu.VMEM((_FLUSH * _D,), f32)] * 2
            + [pltpu.SemaphoreType.DMA] * _DEPTH
        ),
        compiler_params=pltpu.CompilerParams(
            use_tc_tiling_on_sc=True,
            needs_layout_passes=False,
        ),
    )
    def sc_gather(tu, lu, ti, li, ugt, igt, umt, imt,
                  oug, oum, oig, oim, *scratch):
        tuv, luv, tiv, liv = scratch[:4]
        ring = scratch[4:4 + _DEPTH * _GRP * 2]
        outa, outb = scratch[4 + _DEPTH * _GRP * 2: 6 + _DEPTH * _GRP * 2]
        sems = scratch[6 + _DEPTH * _GRP * 2:]
        wid = lax.axis_index("s") * nc + lax.axis_index("c")
        base = wid * bpw
        iota = lax.iota(i32, 16)

        def scalar_at(vec_ref, k):
            v = vec_ref[pl.ds((k // 16) * 16, 16)]
            return jnp.sum(jnp.where(iota == (k % 16), v, 0))

        def do_side(tv_ref, lv_ref, taba, tabb, oha, ohb):
            def slot(d, p):
                return ring[(d * _GRP + p) * 2], ring[(d * _GRP + p) * 2 + 1]

            def issue(g, d):
                for p in range(_GRP):
                    k = g * _GRP + p
                    t = scalar_at(tv_ref, k)
                    c0 = pl.multiple_of(t * _LANES, _LANES)
                    ba, bb = slot(d, p)
                    pltpu.async_copy(taba.at[:, pl.ds(c0, _LANES)], ba, sems[d])
                    pltpu.async_copy(tabb.at[:, pl.ds(c0, _LANES)], bb, sems[d])

            def drain(d):
                for p in range(_GRP):
                    ba, bb = slot(d, p)
                    pltpu.make_async_copy(
                        taba.at[:, pl.ds(0, _LANES)], ba, sems[d]).wait()
                    pltpu.make_async_copy(
                        tabb.at[:, pl.ds(0, _LANES)], bb, sems[d]).wait()

            def extract(g, d):
                for p in range(_GRP):
                    k = g * _GRP + p
                    lane = scalar_at(lv_ref, k)
                    cols = jnp.full((16,), 0, i32) + lane
                    o0 = (k % _FLUSH) * _D
                    ba, bb = slot(d, p)
                    outa[pl.ds(o0, 16)] = plsc.load_gather(ba, [iota, cols])
                    outa[pl.ds(o0 + 16, 16)] = plsc.load_gather(
                        ba, [iota + 16, cols])
                    outb[pl.ds(o0, 16)] = plsc.load_gather(bb, [iota, cols])
                    outb[pl.ds(o0 + 16, 16)] = plsc.load_gather(
                        bb, [iota + 16, cols])

            fpb = _FLUSH * _D
            gpb = _FLUSH // _GRP

            def flush(g):
                blk = g // gpb
                pltpu.sync_copy(outa, oha.at[pl.ds(base * _D + blk * fpb, fpb)])
                pltpu.sync_copy(outb, ohb.at[pl.ds(base * _D + blk * fpb, fpb)])

            for g in range(_DEPTH - 1):
                issue(g, g)

            def body(sg, _):
                for d in range(_DEPTH):
                    g = sg * _DEPTH + d
                    @pl.when(g + _DEPTH - 1 < ngrp)
                    def _():
                        issue(g + _DEPTH - 1, (d + _DEPTH - 1) % _DEPTH)
                    drain(d)
                    extract(g, d)
                    @pl.when(g % gpb == gpb - 1)
                    def _():
                        flush(g)
                return ()

            lax.fori_loop(0, ngrp // _DEPTH, body, (), unroll=1)

        pltpu.sync_copy(tu.at[pl.ds(base, bpw)], tuv)
        pltpu.sync_copy(lu.at[pl.ds(base, bpw)], luv)
        pltpu.sync_copy(ti.at[pl.ds(base, bpw)], tiv)
        pltpu.sync_copy(li.at[pl.ds(base, bpw)], liv)
        do_side(tuv, luv, ugt, umt, oug, oum)
        do_side(tiv, liv, igt, imt, oig, oim)

    return sc_gather


def _tc_body(ug, ig, um, im, w1u, w1i, b1, w2, b2, w3, b3, wog, woh, bo, out):
    f32 = jnp.float32
    gmf = ug[...] * ig[...]
    h = jnp.dot(um[...], w1u[...], preferred_element_type=f32)
    h = h + jnp.dot(im[...], w1i[...], preferred_element_type=f32)
    h = jnp.maximum(h + b1[...], 0.0)
    h = jnp.maximum(jnp.dot(h, w2[...], preferred_element_type=f32) + b2[...], 0.0)
    h = jnp.maximum(jnp.dot(h, w3[...], preferred_element_type=f32) + b3[...], 0.0)
    logits = jnp.sum(gmf * wog[...][None, :], axis=1)
    logits = logits + jnp.sum(h * woh[...][None, :], axis=1)
    logits = logits + bo[...]
    out[...] = 1.0 / (1.0 + jnp.exp(-logits))


def _tc_tail(ug, ig, um, im, w1u, w1i, b1, w2, b2, w3, b3, wog, woh, bo):
    nblk = ug.shape[0] // _BLK
    full2d = lambda shape: pl.BlockSpec(shape, lambda i: (0, 0))
    full1d = lambda n: pl.BlockSpec((n,), lambda i: (0,))
    emb = pl.BlockSpec((_BLK, _D), lambda i: (i, 0))
    return pl.pallas_call(
        _tc_body,
        grid=(nblk,),
        in_specs=[
            emb, emb, emb, emb,
            full2d((_D, 32)), full2d((_D, 32)), full1d(32),
            full2d((32, 16)), full1d(16),
            full2d((16, 8)), full1d(8),
            full1d(_D), full1d(8), full1d(1),
        ],
        out_specs=pl.BlockSpec((_BLK,), lambda i: (i,)),
        out_shape=jax.ShapeDtypeStruct((ug.shape[0],), jnp.float32),
    )(ug, ig, um, im, w1u, w1i, b1, w2, b2, w3, b3, wog, woh, bo)


def kernel(user_ids, item_ids, user_emb_gmf, item_emb_gmf, user_emb_mlp,
           item_emb_mlp, W1, b1, W2, b2, W3, b3, Wo, bo):
    uids = user_ids.astype(jnp.int32)
    iids = item_ids.astype(jnp.int32)
    w1u, w1i = W1[:_D], W1[_D:]
    wog, woh = Wo[:_D, 0], Wo[_D:, 0]
    tabs = (user_emb_gmf.T, item_emb_gmf.T, user_emb_mlp.T, item_emb_mlp.T)
    half = _B // 2
    sc_gather = _make_sc_gather(half)
    parts = []
    gathered = []
    for h in range(2):
        u = lax.slice(uids, (h * half,), ((h + 1) * half,))
        i = lax.slice(iids, (h * half,), ((h + 1) * half,))
        gathered.append(sc_gather(
            u // _LANES, u % _LANES, i // _LANES, i % _LANES, *tabs))
    for h in range(2):
        oug, oum, oig, oim = gathered[h]
        parts.append(_tc_tail(
            oug.reshape(half, _D), oig.reshape(half, _D),
            oum.reshape(half, _D), oim.reshape(half, _D),
            w1u, w1i, b1, W2, b2, W3, b3, wog, woh, bo))
    return jnp.concatenate(parts)


# speedup vs baseline: 1.0199x; 1.0199x over previous
import functools

import jax
import jax.numpy as jnp
from jax import lax
from jax.experimental import pallas as pl
from jax.experimental.pallas import tpu as pltpu
from jax.experimental.pallas import tpu_sc as plsc

_B = 16384
_D = 32
_BLK = 2048
_LANES = 128
_GRP = 2
_DEPTH = 4
_FLUSH = 128


@functools.cache
def _make_sc_gather():
    info = plsc.get_sparse_core_info()
    nc, ns = info.num_cores, info.num_subcores
    nw = nc * ns
    bpw = _B // nw
    ngrp = bpw // _GRP
    fpw = bpw * _D
    mesh = plsc.VectorSubcoreMesh(core_axis_name="c", subcore_axis_name="s")
    f32 = jnp.float32
    i32 = jnp.int32

    slab = pltpu.VMEM((_D, _LANES), f32)
    @functools.partial(
        pl.kernel,
        mesh=mesh,
        out_type=[jax.ShapeDtypeStruct((_B * _D,), f32)] * 4,
        scratch_types=(
            [pltpu.VMEM((bpw,), i32)] * 4
            + [slab] * (_DEPTH * _GRP * 2)
            + [pltpu.VMEM((_FLUSH * _D,), f32)] * 2
            + [pltpu.SemaphoreType.DMA] * _DEPTH
        ),
        compiler_params=pltpu.CompilerParams(
            use_tc_tiling_on_sc=True,
            needs_layout_passes=False,
        ),
    )
    def sc_gather(tu, lu, ti, li, ugt, igt, umt, imt,
                  oug, oum, oig, oim, *scratch):
        tuv, luv, tiv, liv = scratch[:4]
        ring = scratch[4:4 + _DEPTH * _GRP * 2]
        outa, outb = scratch[4 + _DEPTH * _GRP * 2: 6 + _DEPTH * _GRP * 2]
        sems = scratch[6 + _DEPTH * _GRP * 2:]
        wid = lax.axis_index("s") * nc + lax.axis_index("c")
        base = wid * bpw
        iota = lax.iota(i32, 16)

        def scalar_at(vec_ref, k):
            v = vec_ref[pl.ds((k // 16) * 16, 16)]
            return jnp.sum(jnp.where(iota == (k % 16), v, 0))

        def do_side(tv_ref, lv_ref, taba, tabb, oha, ohb):
            def slot(d, p):
                return ring[(d * _GRP + p) * 2], ring[(d * _GRP + p) * 2 + 1]

            def issue(g, d):
                for p in range(_GRP):
                    k = g * _GRP + p
                    t = scalar_at(tv_ref, k)
                    c0 = pl.multiple_of(t * _LANES, _LANES)
                    ba, bb = slot(d, p)
                    pltpu.async_copy(taba.at[:, pl.ds(c0, _LANES)], ba, sems[d])
                    pltpu.async_copy(tabb.at[:, pl.ds(c0, _LANES)], bb, sems[d])

            def drain(d):
                for p in range(_GRP):
                    ba, bb = slot(d, p)
                    pltpu.make_async_copy(
                        taba.at[:, pl.ds(0, _LANES)], ba, sems[d]).wait()
                    pltpu.make_async_copy(
                        tabb.at[:, pl.ds(0, _LANES)], bb, sems[d]).wait()

            def extract(g, d):
                for p in range(_GRP):
                    k = g * _GRP + p
                    lane = scalar_at(lv_ref, k)
                    cols = jnp.full((16,), 0, i32) + lane
                    o0 = (k % _FLUSH) * _D
                    ba, bb = slot(d, p)
                    outa[pl.ds(o0, 16)] = plsc.load_gather(ba, [iota, cols])
                    outa[pl.ds(o0 + 16, 16)] = plsc.load_gather(
                        ba, [iota + 16, cols])
                    outb[pl.ds(o0, 16)] = plsc.load_gather(bb, [iota, cols])
                    outb[pl.ds(o0 + 16, 16)] = plsc.load_gather(
                        bb, [iota + 16, cols])

            fpb = _FLUSH * _D
            gpb = _FLUSH // _GRP

            def flush(g):
                blk = g // gpb
                pltpu.sync_copy(outa, oha.at[pl.ds(base * _D + blk * fpb, fpb)])
                pltpu.sync_copy(outb, ohb.at[pl.ds(base * _D + blk * fpb, fpb)])

            for g in range(_DEPTH - 1):
                issue(g, g)

            def body(sg, _):
                for d in range(_DEPTH):
                    g = sg * _DEPTH + d
                    @pl.when(g + _DEPTH - 1 < ngrp)
                    def _():
                        issue(g + _DEPTH - 1, (d + _DEPTH - 1) % _DEPTH)
                    drain(d)
                    extract(g, d)
                    @pl.when(g % gpb == gpb - 1)
                    def _():
                        flush(g)
                return ()

            lax.fori_loop(0, ngrp // _DEPTH, body, (), unroll=1)

        pltpu.sync_copy(tu.at[pl.ds(base, bpw)], tuv)
        pltpu.sync_copy(lu.at[pl.ds(base, bpw)], luv)
        pltpu.sync_copy(ti.at[pl.ds(base, bpw)], tiv)
        pltpu.sync_copy(li.at[pl.ds(base, bpw)], liv)
        do_side(tuv, luv, ugt, umt, oug, oum)
        do_side(tiv, liv, igt, imt, oig, oim)

    return sc_gather


def _tc_body(ug, ig, um, im, w1u, w1i, b1, w2, b2, w3, b3, wog, woh, bo, out):
    f32 = jnp.float32
    gmf = ug[...] * ig[...]
    h = jnp.dot(um[...], w1u[...], preferred_element_type=f32)
    h = h + jnp.dot(im[...], w1i[...], preferred_element_type=f32)
    h = jnp.maximum(h + b1[...], 0.0)
    h = jnp.maximum(jnp.dot(h, w2[...], preferred_element_type=f32) + b2[...], 0.0)
    h = jnp.maximum(jnp.dot(h, w3[...], preferred_element_type=f32) + b3[...], 0.0)
    logits = jnp.sum(gmf * wog[...][None, :], axis=1)
    logits = logits + jnp.sum(h * woh[...][None, :], axis=1)
    logits = logits + bo[...]
    out[...] = 1.0 / (1.0 + jnp.exp(-logits))


def _tc_tail(ug, ig, um, im, w1u, w1i, b1, w2, b2, w3, b3, wog, woh, bo):
    nblk = _B // _BLK
    full2d = lambda shape: pl.BlockSpec(shape, lambda i: (0, 0))
    full1d = lambda n: pl.BlockSpec((n,), lambda i: (0,))
    emb = pl.BlockSpec((_BLK, _D), lambda i: (i, 0))
    return pl.pallas_call(
        _tc_body,
        grid=(nblk,),
        in_specs=[
            emb, emb, emb, emb,
            full2d((_D, 32)), full2d((_D, 32)), full1d(32),
            full2d((32, 16)), full1d(16),
            full2d((16, 8)), full1d(8),
            full1d(_D), full1d(8), full1d(1),
        ],
        out_specs=pl.BlockSpec((_BLK,), lambda i: (i,)),
        out_shape=jax.ShapeDtypeStruct((_B,), jnp.float32),
    )(ug, ig, um, im, w1u, w1i, b1, w2, b2, w3, b3, wog, woh, bo)


def kernel(user_ids, item_ids, user_emb_gmf, item_emb_gmf, user_emb_mlp,
           item_emb_mlp, W1, b1, W2, b2, W3, b3, Wo, bo):
    uids = user_ids.astype(jnp.int32)
    iids = item_ids.astype(jnp.int32)
    sc_gather = _make_sc_gather()
    oug, oum, oig, oim = sc_gather(
        uids // _LANES, uids % _LANES, iids // _LANES, iids % _LANES,
        user_emb_gmf.T, item_emb_gmf.T, user_emb_mlp.T, item_emb_mlp.T)
    ug = oug.reshape(_B, _D)
    um = oum.reshape(_B, _D)
    ig = oig.reshape(_B, _D)
    im = oim.reshape(_B, _D)
    w1u, w1i = W1[:_D], W1[_D:]
    wog, woh = Wo[:_D, 0], Wo[_D:, 0]
    return _tc_tail(ug, ig, um, im, w1u, w1i, b1, W2, b2, W3, b3, wog, woh, bo)
